# Initial kernel scaffold; baseline (speedup 1.0000x reference)
import jax
import jax.numpy as jnp
from jax.experimental import pallas as pl


def kernel(x, Wg, W1, W2, W3):
    raise NotImplementedError("write your pallas kernel here")


# speedup vs baseline: 1.5934x; 1.5934x over previous
import functools

import jax
import jax.numpy as jnp
from jax import lax
from jax.experimental import pallas as pl
from jax.experimental.pallas import tpu as pltpu

N = 8192
D = 1024
F = 4096
E = 8
K = 2

BM = 256
BF = 1024
NF = F // BF
NB = 72
M_PAD = NB * BM

BR = 1024
NEG = -1e30


def _router_body(x_ref, wg_ref, idx_ref, w_ref):
    xb = x_ref[...]
    g = jnp.dot(xb, wg_ref[...], preferred_element_type=jnp.float32)
    lane = lax.broadcasted_iota(jnp.int32, g.shape, 1)
    valid = lane < E
    gm = jnp.where(valid, g, NEG)
    m1 = jnp.max(gm, axis=1, keepdims=True)
    i1 = jnp.min(jnp.where(gm == m1, lane, 999), axis=1, keepdims=True)
    g2 = jnp.where(lane == i1, NEG, gm)
    m2 = jnp.max(g2, axis=1, keepdims=True)
    i2 = jnp.min(jnp.where(g2 == m2, lane, 999), axis=1, keepdims=True)
    e2 = jnp.exp(m2 - m1)
    w1 = 1.0 / (1.0 + e2)
    w2 = e2 * w1
    lane8 = lax.broadcasted_iota(jnp.int32, (BR, E), 1)
    idx_ref[...] = jnp.where(lane8 == 0, i1, i2)
    w_ref[...] = jnp.where(lane8 == 0, w1, w2)


def _router(x_flat, Wg):
    wg_pad = jnp.zeros((D, 128), jnp.float32).at[:, :E].set(Wg)
    return pl.pallas_call(
        _router_body,
        grid=(N // BR,),
        in_specs=[
            pl.BlockSpec((BR, D), lambda i: (i, 0)),
            pl.BlockSpec((D, 128), lambda i: (0, 0)),
        ],
        out_specs=[
            pl.BlockSpec((BR, E), lambda i: (i, 0)),
            pl.BlockSpec((BR, E), lambda i: (i, 0)),
        ],
        out_shape=[
            jax.ShapeDtypeStruct((N, E), jnp.int32),
            jax.ShapeDtypeStruct((N, E), jnp.float32),
        ],
    )(x_flat, wg_pad)


def _metadata(top_idx, top_w):
    e_flat = top_idx.reshape(-1)
    w_flat = top_w.reshape(-1)
    order = jnp.argsort(e_flat, stable=True)
    e_sorted = e_flat[order]
    offs = jnp.searchsorted(e_sorted, jnp.arange(E, dtype=e_sorted.dtype),
                            side="left").astype(jnp.int32)
    counts = jnp.diff(jnp.concatenate([offs, jnp.array([N * K], jnp.int32)]))
    nb_e = (counts + BM - 1) // BM
    blk_end = jnp.cumsum(nb_e).astype(jnp.int32)
    blk_start = blk_end - nb_e
    row_start = blk_start * BM
    s = jnp.arange(N * K, dtype=jnp.int32)
    pos_s = row_start[e_sorted] + (s - offs[e_sorted])
    gather_idx = jnp.zeros((M_PAD,), jnp.int32).at[pos_s].set(
        (order // K).astype(jnp.int32))
    w_pad = jnp.zeros((M_PAD,), jnp.float32).at[pos_s].set(w_flat[order])
    inv_pos = jnp.zeros((N * K,), jnp.int32).at[order].set(pos_s)
    block_expert = jnp.minimum(
        jnp.searchsorted(blk_end, jnp.arange(NB, dtype=jnp.int32),
                         side="right").astype(jnp.int32), E - 1)
    return gather_idx, w_pad, inv_pos, block_expert


def _ffn_body(be_ref, xs_ref, w1_ref, w3_ref, w2_ref, wrow_ref, out_ref):
    j = pl.program_id(1)
    xb = xs_ref[...].astype(jnp.bfloat16)
    a = jnp.dot(xb, w1_ref[0], preferred_element_type=jnp.float32)
    c = jnp.dot(xb, w3_ref[0], preferred_element_type=jnp.float32)
    h = (a * jax.nn.sigmoid(a) * c).astype(jnp.bfloat16)
    part = jnp.dot(h, w2_ref[0], preferred_element_type=jnp.float32)

    @pl.when(j == 0)
    def _():
        out_ref[...] = jnp.zeros_like(out_ref)

    out_ref[...] += part

    @pl.when(j == NF - 1)
    def _():
        out_ref[...] *= wrow_ref[:, 0:1]


def _grouped_ffn(xs, W1b, W3b, W2b, w2d, block_expert):
    grid_spec = pltpu.PrefetchScalarGridSpec(
        num_scalar_prefetch=1,
        grid=(NB, NF),
        in_specs=[
            pl.BlockSpec((BM, D), lambda i, j, be: (i, 0)),
            pl.BlockSpec((1, D, BF), lambda i, j, be: (be[i], 0, j)),
            pl.BlockSpec((1, D, BF), lambda i, j, be: (be[i], 0, j)),
            pl.BlockSpec((1, BF, D), lambda i, j, be: (be[i], j, 0)),
            pl.BlockSpec((BM, 128), lambda i, j, be: (i, 0)),
        ],
        out_specs=pl.BlockSpec((BM, D), lambda i, j, be: (i, 0)),
    )
    return pl.pallas_call(
        _ffn_body,
        grid_spec=grid_spec,
        out_shape=jax.ShapeDtypeStruct((M_PAD, D), jnp.float32),
        compiler_params=pltpu.CompilerParams(
            dimension_semantics=("arbitrary", "arbitrary")),
    )(block_expert, xs, W1b, W3b, W2b, w2d)


def kernel(x, Wg, W1, W2, W3):
    Bb, Tt, Dd = x.shape
    x_flat = x.reshape(-1, Dd)
    top_idx, top_w = _router(x_flat, Wg)
    gather_idx, w_pad, inv_pos, block_expert = _metadata(
        top_idx[:, :K], top_w[:, :K])

    W1b = W1.astype(jnp.bfloat16)
    W3b = W3.astype(jnp.bfloat16)
    W2b = W2.astype(jnp.bfloat16)

    xs = jnp.take(x_flat, gather_idx, axis=0)
    w2d = jnp.broadcast_to(w_pad[:, None], (M_PAD, 128))

    ys = _grouped_ffn(xs, W1b, W3b, W2b, w2d, block_expert)

    pos = inv_pos.reshape(N, K)
    out = jnp.take(ys, pos[:, 0], axis=0) + jnp.take(ys, pos[:, 1], axis=0)
    return out.reshape(Bb, Tt, Dd)
